# baseline (speedup 1.0000x reference)
import jax
import jax.numpy as jnp
from jax import lax
from jax.experimental import pallas as pl
from jax.experimental.pallas import tpu as pltpu
from jax.experimental.pallas import tpu_sc as plsc

VOCAB = 1000000
D = 64
S = 200
B = 4096

NC, NS = 2, 16
NW = NC * NS
BT = B // NW
LANES = 16
DT = D // 8
GRP = 8
CC = D // LANES


def _body(xa_hbm, tok_hbm, pos_hbm, out_hbm,
          idx_v, pos_v, g0, g1, g2, g3, t0, t1, t2, t3,
          gs0, gs1, gs2, gs3, ss0, ss1, ss2, ss3):
    gath = (g0, g1, g2, g3)
    tbuf = (t0, t1, t2, t3)
    g_sems = (gs0, gs1, gs2, gs3)
    s_sems = (ss0, ss1, ss2, ss3)

    wid = lax.axis_index("s") * NC + lax.axis_index("c")
    pltpu.sync_copy(xa_hbm.at[wid], idx_v)
    pltpu.sync_copy(pos_hbm, pos_v)

    iota = lax.iota(jnp.int32, LANES)
    dvec = [16 * cc + iota for cc in range(CC)]
    dbase = [(dv // 8) * 1024 + (dv % 8) * 128 for dv in dvec]

    def fire_gather(s, p):
        pltpu.async_copy(tok_hbm.at[idx_v.at[s]], gath[p], g_sems[p])

    def wait_gather(s, p):
        pltpu.make_async_copy(tok_hbm.at[idx_v.at[s]], gath[p],
                              g_sems[p]).wait()

    def fire_store(s, p):
        for dt in range(DT):
            pltpu.async_copy(tbuf[p].at[pl.ds(dt * 1024, 1024)],
                             out_hbm.at[s, dt, wid], s_sems[p])

    def wait_store(s, p):
        for dt in range(DT):
            pltpu.make_async_copy(tbuf[p].at[pl.ds(dt * 1024, 1024)],
                                  out_hbm.at[s, dt, wid], s_sems[p]).wait()

    def compute(s, p):
        pr = [pos_v[s, pl.ds(16 * cc, LANES)] for cc in range(CC)]

        @pl.loop(0, BT, step=GRP)
        def _grp(b0):
            vals = []
            for r in range(GRP):
                for cc in range(CC):
                    vals.append(gath[p][b0 + r, pl.ds(16 * cc, LANES)])
            sums = []
            for r in range(GRP):
                for cc in range(CC):
                    sums.append(vals[r * CC + cc] + pr[cc])
            for r in range(GRP):
                for cc in range(CC):
                    plsc.store_scatter(tbuf[p], [dbase[cc] + (b0 + r)],
                                       sums[r * CC + cc])

    for k in range(4):
        fire_gather(k, k)

    @pl.loop(0, S, step=4)
    def _s0(s0):
        for par in range(4):
            s = s0 + par
            wait_gather(s, par)

            @pl.when(s >= 4)
            def _():
                wait_store(s - 4, par)

            compute(s, par)
            fire_store(s, par)

            @pl.when(s + 4 < S)
            def _():
                fire_gather(s + 4, par)

    for k in range(4):
        wait_store(S - 4 + k, k)


@jax.jit
def _run(xa, token_emb, pos_emb):
    mesh = plsc.VectorSubcoreMesh(core_axis_name="c", subcore_axis_name="s",
                                  num_cores=NC, num_subcores=NS)
    return pl.kernel(
        _body,
        out_type=jax.ShapeDtypeStruct((S, DT, NW, 8 * BT), jnp.float32),
        mesh=mesh,
        compiler_params=pltpu.CompilerParams(use_tc_tiling_on_sc=False,
                                             needs_layout_passes=False),
        scratch_types=(
            [pltpu.VMEM((S, BT), jnp.int32),
             pltpu.VMEM((S, D), jnp.float32),
             pltpu.VMEM((BT, D), jnp.float32),
             pltpu.VMEM((BT, D), jnp.float32),
             pltpu.VMEM((BT, D), jnp.float32),
             pltpu.VMEM((BT, D), jnp.float32),
             pltpu.VMEM((DT * 8 * BT,), jnp.float32),
             pltpu.VMEM((DT * 8 * BT,), jnp.float32),
             pltpu.VMEM((DT * 8 * BT,), jnp.float32),
             pltpu.VMEM((DT * 8 * BT,), jnp.float32)]
            + [pltpu.SemaphoreType.DMA for _ in range(8)]
        ),
    )(xa, token_emb, pos_emb)


def kernel(x, token_emb, pos_emb):
    xa = x.T.reshape(S, NW, BT).transpose(1, 0, 2)
    out5 = _run(xa, token_emb, pos_emb)
    return (out5.reshape(S, DT, NW, 8, BT)
            .transpose(2, 4, 0, 1, 3).reshape(B, S, D))


# speedup vs baseline: 1.7382x; 1.7382x over previous
import jax
import jax.numpy as jnp
from jax import lax
from jax.experimental import pallas as pl
from jax.experimental.pallas import tpu as pltpu
from jax.experimental.pallas import tpu_sc as plsc

VOCAB = 1000000
D = 64
S = 200
B = 4096

NC, NS = 2, 16
NW = NC * NS
BT = B // NW
LANES = 16
DT = D // 8
GRP = 8
CC = D // LANES


def _body(xa_hbm, tok_hbm, pos_hbm, out_hbm,
          idx_v, pos_v, g0, g1, g2, g3, t0, t1, t2, t3,
          gs0, gs1, gs2, gs3, ss0, ss1, ss2, ss3):
    gath = (g0, g1, g2, g3)
    tbuf = (t0, t1, t2, t3)
    g_sems = (gs0, gs1, gs2, gs3)
    s_sems = (ss0, ss1, ss2, ss3)

    wid = lax.axis_index("s") * NC + lax.axis_index("c")
    pltpu.sync_copy(xa_hbm.at[wid], idx_v)
    pltpu.sync_copy(pos_hbm, pos_v)

    iota = lax.iota(jnp.int32, LANES)
    dvec = [16 * cc + iota for cc in range(CC)]

    def fire_gather(s, p):
        pltpu.async_copy(tok_hbm.at[idx_v.at[s]], gath[p], g_sems[p])

    def wait_gather(s, p):
        pltpu.make_async_copy(tok_hbm.at[idx_v.at[s]], gath[p],
                              g_sems[p]).wait()

    def fire_store(s, p):
        for dt in range(DT):
            pltpu.async_copy(tbuf[p].at[pl.ds(dt * 8, 8), pl.ds(0, BT)],
                             out_hbm.at[s, dt, wid], s_sems[p])

    def wait_store(s, p):
        for dt in range(DT):
            pltpu.make_async_copy(tbuf[p].at[pl.ds(dt * 8, 8), pl.ds(0, BT)],
                                  out_hbm.at[s, dt, wid], s_sems[p]).wait()

    def compute(s, p):
        pr = [pos_v[s, pl.ds(16 * cc, LANES)] for cc in range(CC)]

        @pl.loop(0, BT, step=GRP)
        def _grp(b0):
            vals = []
            for r in range(GRP):
                for cc in range(CC):
                    vals.append(gath[p][b0 + r, pl.ds(16 * cc, LANES)])
            sums = []
            for r in range(GRP):
                for cc in range(CC):
                    sums.append(vals[r * CC + cc] + pr[cc])
            cols = [(b0 + r) + 0 * iota for r in range(GRP)]
            for r in range(GRP):
                for cc in range(CC):
                    plsc.store_scatter(tbuf[p], [dvec[cc], cols[r]],
                                       sums[r * CC + cc])

    for k in range(4):
        fire_gather(k, k)

    @pl.loop(0, S, step=4)
    def _s0(s0):
        for par in range(4):
            s = s0 + par
            wait_gather(s, par)

            @pl.when(s >= 4)
            def _():
                wait_store(s - 4, par)

            compute(s, par)
            fire_store(s, par)

            @pl.when(s + 4 < S)
            def _():
                fire_gather(s + 4, par)

    for k in range(4):
        wait_store(S - 4 + k, k)


@jax.jit
def _run(xa, token_emb, pos_emb):
    mesh = plsc.VectorSubcoreMesh(core_axis_name="c", subcore_axis_name="s",
                                  num_cores=NC, num_subcores=NS)
    return pl.kernel(
        _body,
        out_type=jax.ShapeDtypeStruct((S, DT, NW, 8, BT), jnp.float32),
        mesh=mesh,
        compiler_params=pltpu.CompilerParams(use_tc_tiling_on_sc=False,
                                             needs_layout_passes=False),
        scratch_types=(
            [pltpu.VMEM((S, BT), jnp.int32),
             pltpu.VMEM((S, D), jnp.float32),
             pltpu.VMEM((BT, D), jnp.float32),
             pltpu.VMEM((BT, D), jnp.float32),
             pltpu.VMEM((BT, D), jnp.float32),
             pltpu.VMEM((BT, D), jnp.float32),
             pltpu.VMEM((D, BT + 1), jnp.float32),
             pltpu.VMEM((D, BT + 1), jnp.float32),
             pltpu.VMEM((D, BT + 1), jnp.float32),
             pltpu.VMEM((D, BT + 1), jnp.float32)]
            + [pltpu.SemaphoreType.DMA for _ in range(8)]
        ),
    )(xa, token_emb, pos_emb)


def kernel(x, token_emb, pos_emb):
    xa = x.T.reshape(S, NW, BT).transpose(1, 0, 2)
    out5 = _run(xa, token_emb, pos_emb)
    return out5.transpose(2, 4, 0, 1, 3).reshape(B, S, D)
